# Initial kernel scaffold; baseline (speedup 1.0000x reference)
import jax
import jax.numpy as jnp
from jax.experimental import pallas as pl


def kernel(x, edge_index, batch, W1, b1, g1, be1, W2, b2, g2, be2, W3, b3, Wl, bl):
    raise NotImplementedError("write your pallas kernel here")


# speedup vs baseline: 7.1436x; 7.1436x over previous
import functools

import jax
import jax.numpy as jnp
from jax import lax
from jax.experimental import pallas as pl
from jax.experimental.pallas import tpu as pltpu
from jax.experimental.pallas import tpu_sc as plsc

N = 10000
D = 128
G = 64
NC = 2
NS = 16
L = 16
NW = NC * NS
CH = 128
NPAD = 10240
RPT = NPAD // NS

_mesh = plsc.VectorSubcoreMesh(core_axis_name="c", subcore_axis_name="s")


def _make_deg(rpw):
    @functools.partial(
        pl.kernel,
        out_type=jax.ShapeDtypeStruct((NC, NPAD, D), jnp.float32),
        mesh=_mesh,
        scratch_types=[
            pltpu.VMEM((rpw, CH), jnp.int32),
            pltpu.VMEM((CH, D), jnp.float32),
            pltpu.VMEM_SHARED((NPAD, D), jnp.float32),
        ],
    )
    def deg_kernel(dst_hbm, ones_hbm, zer_hbm, out_hbm, dst_v, ones_v, acc_sh):
        c = lax.axis_index("c")
        s = lax.axis_index("s")
        w = c * NS + s
        sl = pl.ds(s * RPT, RPT)
        pltpu.sync_copy(zer_hbm, acc_sh.at[sl])
        pltpu.sync_copy(ones_hbm, ones_v)
        pltpu.sync_copy(dst_hbm.at[pl.ds(w * rpw, rpw)], dst_v)
        plsc.subcore_barrier()

        def body(j, carry):
            pltpu.sync_copy(ones_v, acc_sh.at[dst_v.at[j]], add=True)
            return carry

        lax.fori_loop(0, rpw, body, 0)
        plsc.subcore_barrier()
        pltpu.sync_copy(acc_sh.at[sl], out_hbm.at[c].at[sl])

    return deg_kernel


def _make_aggr(rpw):
    @functools.partial(
        pl.kernel,
        out_type=jax.ShapeDtypeStruct((NC, NPAD, D), jnp.float32),
        mesh=_mesh,
        scratch_types=[
            pltpu.VMEM((rpw, CH), jnp.int32),
            pltpu.VMEM((rpw, CH), jnp.int32),
            pltpu.VMEM((CH, D), jnp.float32),
            pltpu.VMEM_SHARED((NPAD, D), jnp.float32),
            pltpu.SemaphoreType.DMA,
        ],
    )
    def aggr_kernel(hp_hbm, src_hbm, dst_hbm, zer_hbm, out_hbm,
                    src_v, dst_v, buf, acc_sh, sem):
        c = lax.axis_index("c")
        s = lax.axis_index("s")
        w = c * NS + s
        sl = pl.ds(s * RPT, RPT)
        pltpu.sync_copy(zer_hbm, acc_sh.at[sl])
        pltpu.sync_copy(src_hbm.at[pl.ds(w * rpw, rpw)], src_v)
        pltpu.sync_copy(dst_hbm.at[pl.ds(w * rpw, rpw)], dst_v)
        plsc.subcore_barrier()

        def body(j, carry):
            pltpu.async_copy(hp_hbm.at[src_v.at[j]], buf, sem).wait()
            pltpu.sync_copy(buf, acc_sh.at[dst_v.at[j]], add=True)
            return carry

        lax.fori_loop(0, rpw, body, 0)
        plsc.subcore_barrier()
        pltpu.sync_copy(acc_sh.at[sl], out_hbm.at[c].at[sl])

    return aggr_kernel


def _tc_stage1(x_ref, w_ref, deg_ref, hp_ref, dinv_ref):
    deg = deg_ref[0, :N, 0:1] + deg_ref[1, :N, 0:1]
    dinv = lax.rsqrt(deg + 1.0)
    h = jnp.dot(x_ref[...], w_ref[...], preferred_element_type=jnp.float32)
    hp_ref[...] = h * dinv
    dinv_ref[...] = dinv


def _tc_mid(acc_ref, hp_ref, dinv_ref, b_ref, g_ref, be_ref, w_ref, out_ref):
    accsum = acc_ref[0, :N, :] + acc_ref[1, :N, :] + hp_ref[...]
    pre = accsum * dinv_ref[...] + b_ref[...]
    mu = jnp.mean(pre, axis=0, keepdims=True)
    var = jnp.mean((pre - mu) ** 2, axis=0, keepdims=True)
    y = jnp.maximum(g_ref[...] * (pre - mu) * lax.rsqrt(var + 1e-5)
                    + be_ref[...], 0.0)
    out_ref[...] = jnp.dot(y, w_ref[...],
                           preferred_element_type=jnp.float32) * dinv_ref[...]


def _tc_fin(acc_ref, hp_ref, dinv_ref, b_ref, batch_ref, wl_ref, bl_ref,
            out_ref):
    pre = (acc_ref[0, :N, :] + acc_ref[1, :N, :] + hp_ref[...]) \
        * dinv_ref[...] + b_ref[...]
    h = jnp.maximum(pre, 0.0)
    seg = lax.broadcasted_iota(jnp.int32, (G, N), 0)
    onehot = (seg == batch_ref[...]).astype(jnp.float32)
    pooled = jnp.dot(onehot, h, preferred_element_type=jnp.float32)
    counts = jnp.sum(onehot, axis=1, keepdims=True)
    pooled = pooled / jnp.maximum(counts, 1.0)
    out_ref[...] = jnp.dot(pooled, wl_ref[...],
                           preferred_element_type=jnp.float32) + bl_ref[...]


_f32 = jnp.float32

_stage1 = pl.pallas_call(
    _tc_stage1,
    out_shape=(jax.ShapeDtypeStruct((N, D), _f32),
               jax.ShapeDtypeStruct((N, 1), _f32)),
)

def _mid_call(acc, hp, dinv, b, g, be, w):
    return pl.pallas_call(
        _tc_mid,
        out_shape=jax.ShapeDtypeStruct((N, D), _f32),
    )(acc, hp, dinv, b, g, be, w)


def _fin_call(acc, hp, dinv, b, batch2d, wl, bl):
    return pl.pallas_call(
        _tc_fin,
        out_shape=jax.ShapeDtypeStruct((G, 1), _f32),
    )(acc, hp, dinv, b, batch2d, wl, bl)


def kernel(x, edge_index, batch, W1, b1, g1, be1, W2, b2, g2, be2, W3, b3,
           Wl, bl):
    E = edge_index.shape[1]
    rpw = -(-E // (NW * CH))
    rpw = -(-rpw // 8) * 8
    epad = NW * rpw * CH
    pad = epad - E
    src = jnp.concatenate([edge_index[0], jnp.zeros((pad,), jnp.int32)])
    dst = jnp.concatenate([edge_index[1], jnp.full((pad,), N, jnp.int32)])
    src2d = src.reshape(NW * rpw, CH)
    dst2d = dst.reshape(NW * rpw, CH)
    zer = jnp.zeros((RPT, D), _f32)

    deg = _make_deg(rpw)(dst2d, jnp.ones((CH, D), _f32), zer)

    aggr = _make_aggr(rpw)

    hp1, dinv = _stage1(x, W1, deg)
    acc1 = aggr(hp1, src2d, dst2d, zer)
    hp2 = _mid_call(acc1, hp1, dinv, b1.reshape(1, D), g1.reshape(1, D),
                    be1.reshape(1, D), W2)
    acc2 = aggr(hp2, src2d, dst2d, zer)
    hp3 = _mid_call(acc2, hp2, dinv, b2.reshape(1, D), g2.reshape(1, D),
                    be2.reshape(1, D), W3)
    acc3 = aggr(hp3, src2d, dst2d, zer)
    return _fin_call(acc3, hp3, dinv, b3.reshape(1, D), batch.reshape(1, N),
                     Wl, bl.reshape(1, 1))


# speedup vs baseline: 7.1456x; 1.0003x over previous
import functools

import jax
import jax.numpy as jnp
from jax import lax
from jax.experimental import pallas as pl
from jax.experimental.pallas import tpu as pltpu
from jax.experimental.pallas import tpu_sc as plsc

N = 10000
D = 128
G = 64
NC = 2
NS = 16
L = 16
NW = NC * NS
CH = 128
NPAD = 10240
RPT = NPAD // NS

_mesh = plsc.VectorSubcoreMesh(core_axis_name="c", subcore_axis_name="s")


def _make_deg(rpw):
    @functools.partial(
        pl.kernel,
        out_type=jax.ShapeDtypeStruct((NC, NPAD, D), jnp.float32),
        mesh=_mesh,
        scratch_types=[
            pltpu.VMEM((rpw, CH), jnp.int32),
            pltpu.VMEM((CH, D), jnp.float32),
            pltpu.VMEM_SHARED((NPAD, D), jnp.float32),
        ],
    )
    def deg_kernel(dst_hbm, ones_hbm, zer_hbm, out_hbm, dst_v, ones_v, acc_sh):
        c = lax.axis_index("c")
        s = lax.axis_index("s")
        w = c * NS + s
        sl = pl.ds(s * RPT, RPT)
        pltpu.sync_copy(zer_hbm, acc_sh.at[sl])
        pltpu.sync_copy(ones_hbm, ones_v)
        pltpu.sync_copy(dst_hbm.at[pl.ds(w * rpw, rpw)], dst_v)
        plsc.subcore_barrier()

        def body(j, carry):
            pltpu.sync_copy(ones_v, acc_sh.at[dst_v.at[j]], add=True)
            return carry

        lax.fori_loop(0, rpw, body, 0)
        plsc.subcore_barrier()
        pltpu.sync_copy(acc_sh.at[sl], out_hbm.at[c].at[sl])

    return deg_kernel


def _make_aggr(rpw):
    @functools.partial(
        pl.kernel,
        out_type=jax.ShapeDtypeStruct((NC, NPAD, D), jnp.float32),
        mesh=_mesh,
        scratch_types=[
            pltpu.VMEM((rpw, CH), jnp.int32),
            pltpu.VMEM((rpw, CH), jnp.int32),
            pltpu.VMEM((CH, D), jnp.float32),
            pltpu.VMEM_SHARED((NPAD, D), jnp.float32),
            pltpu.SemaphoreType.DMA,
        ],
    )
    def aggr_kernel(hp_hbm, src_hbm, dst_hbm, zer_hbm, out_hbm,
                    src_v, dst_v, buf, acc_sh, sem):
        c = lax.axis_index("c")
        s = lax.axis_index("s")
        w = c * NS + s
        sl = pl.ds(s * RPT, RPT)
        pltpu.sync_copy(zer_hbm, acc_sh.at[sl])
        pltpu.sync_copy(src_hbm.at[pl.ds(w * rpw, rpw)], src_v)
        pltpu.sync_copy(dst_hbm.at[pl.ds(w * rpw, rpw)], dst_v)
        plsc.subcore_barrier()

        def body(j, carry):
            pltpu.async_copy(hp_hbm.at[src_v.at[j]], buf, sem).wait()
            pltpu.sync_copy(buf, acc_sh.at[dst_v.at[j]], add=True)
            return carry

        lax.fori_loop(0, rpw, body, 0)
        plsc.subcore_barrier()
        pltpu.sync_copy(acc_sh.at[sl], out_hbm.at[c].at[sl])

    return aggr_kernel


def _tc_stage1(x_ref, w_ref, deg_ref, hp_ref, dinv_ref):
    deg = deg_ref[0, :N, 0:1] + deg_ref[1, :N, 0:1]
    dinv = lax.rsqrt(deg + 1.0)
    h = jnp.dot(x_ref[...], w_ref[...], preferred_element_type=jnp.float32)
    hp_ref[...] = h * dinv
    dinv_ref[...] = dinv


def _tc_mid(acc_ref, hp_ref, dinv_ref, b_ref, g_ref, be_ref, w_ref, out_ref):
    accsum = acc_ref[0, :N, :] + acc_ref[1, :N, :] + hp_ref[...]
    pre = accsum * dinv_ref[...] + b_ref[...]
    mu = jnp.mean(pre, axis=0, keepdims=True)
    var = jnp.mean((pre - mu) ** 2, axis=0, keepdims=True)
    y = jnp.maximum(g_ref[...] * (pre - mu) * lax.rsqrt(var + 1e-5)
                    + be_ref[...], 0.0)
    out_ref[...] = jnp.dot(y, w_ref[...],
                           preferred_element_type=jnp.float32) * dinv_ref[...]


def _tc_fin(acc_ref, hp_ref, dinv_ref, b_ref, batch_ref, wl_ref, bl_ref,
            out_ref):
    pre = (acc_ref[0, :N, :] + acc_ref[1, :N, :] + hp_ref[...]) \
        * dinv_ref[...] + b_ref[...]
    h = jnp.maximum(pre, 0.0)
    seg = lax.broadcasted_iota(jnp.int32, (G, N), 0)
    onehot = (seg == batch_ref[...]).astype(jnp.float32)
    pooled = jnp.dot(onehot, h, preferred_element_type=jnp.float32)
    counts = jnp.sum(onehot, axis=1, keepdims=True)
    pooled = pooled / jnp.maximum(counts, 1.0)
    out_ref[...] = jnp.dot(pooled, wl_ref[...],
                           preferred_element_type=jnp.float32) + bl_ref[...]


_f32 = jnp.float32

_stage1 = pl.pallas_call(
    _tc_stage1,
    out_shape=(jax.ShapeDtypeStruct((N, D), _f32),
               jax.ShapeDtypeStruct((N, 1), _f32)),
)

def _mid_call(acc, hp, dinv, b, g, be, w):
    return pl.pallas_call(
        _tc_mid,
        out_shape=jax.ShapeDtypeStruct((N, D), _f32),
    )(acc, hp, dinv, b, g, be, w)


def _fin_call(acc, hp, dinv, b, batch2d, wl, bl):
    return pl.pallas_call(
        _tc_fin,
        out_shape=jax.ShapeDtypeStruct((G, 1), _f32),
    )(acc, hp, dinv, b, batch2d, wl, bl)


def kernel(x, edge_index, batch, W1, b1, g1, be1, W2, b2, g2, be2, W3, b3,
           Wl, bl):
    E = edge_index.shape[1]
    rpw = -(-E // (NW * CH))
    rpw = -(-rpw // 8) * 8
    epad = NW * rpw * CH
    pad = epad - E
    src = jnp.concatenate([edge_index[0], jnp.zeros((pad,), jnp.int32)])
    trash = N + jnp.arange(pad, dtype=jnp.int32) % (NPAD - N)
    dst = jnp.concatenate([edge_index[1], trash])
    src2d = src.reshape(NW * rpw, CH)
    dst2d = dst.reshape(NW * rpw, CH)
    zer = jnp.zeros((RPT, D), _f32)

    deg = _make_deg(rpw)(dst2d, jnp.ones((CH, D), _f32), zer)

    aggr = _make_aggr(rpw)

    hp1, dinv = _stage1(x, W1, deg)
    acc1 = aggr(hp1, src2d, dst2d, zer)
    hp2 = _mid_call(acc1, hp1, dinv, b1.reshape(1, D), g1.reshape(1, D),
                    be1.reshape(1, D), W2)
    acc2 = aggr(hp2, src2d, dst2d, zer)
    hp3 = _mid_call(acc2, hp2, dinv, b2.reshape(1, D), g2.reshape(1, D),
                    be2.reshape(1, D), W3)
    acc3 = aggr(hp3, src2d, dst2d, zer)
    return _fin_call(acc3, hp3, dinv, b3.reshape(1, D), batch.reshape(1, N),
                     Wl, bl.reshape(1, 1))


# speedup vs baseline: 7.8225x; 1.0947x over previous
import functools

import jax
import jax.numpy as jnp
from jax import lax
from jax.experimental import pallas as pl
from jax.experimental.pallas import tpu as pltpu
from jax.experimental.pallas import tpu_sc as plsc

N = 10000
D = 128
G = 64
NC = 2
NS = 16
L = 16
NW = NC * NS
CH = 128
NPAD = 10240
RPT = NPAD // NS

_mesh = plsc.VectorSubcoreMesh(core_axis_name="c", subcore_axis_name="s")


def _make_deg(rpw):
    @functools.partial(
        pl.kernel,
        out_type=jax.ShapeDtypeStruct((NC, NPAD, D), jnp.float32),
        mesh=_mesh,
        scratch_types=[
            pltpu.VMEM((rpw, CH), jnp.int32),
            pltpu.VMEM((CH, D), jnp.float32),
            pltpu.VMEM_SHARED((NPAD, D), jnp.float32),
        ],
    )
    def deg_kernel(dst_hbm, ones_hbm, zer_hbm, out_hbm, dst_v, ones_v, acc_sh):
        c = lax.axis_index("c")
        s = lax.axis_index("s")
        w = c * NS + s
        sl = pl.ds(s * RPT, RPT)
        pltpu.sync_copy(zer_hbm, acc_sh.at[sl])
        pltpu.sync_copy(ones_hbm, ones_v)
        pltpu.sync_copy(dst_hbm.at[pl.ds(w * rpw, rpw)], dst_v)
        plsc.subcore_barrier()

        def body(j, carry):
            pltpu.sync_copy(ones_v, acc_sh.at[dst_v.at[j]], add=True)
            return carry

        lax.fori_loop(0, rpw, body, 0)
        plsc.subcore_barrier()
        pltpu.sync_copy(acc_sh.at[sl], out_hbm.at[c].at[sl])

    return deg_kernel


NBUF = 2
GW = 8


def _make_aggr(rpw):
    @functools.partial(
        pl.kernel,
        out_type=jax.ShapeDtypeStruct((NC, NPAD, D), jnp.float32),
        mesh=_mesh,
        scratch_types=[
            pltpu.VMEM((rpw, CH), jnp.int32),
            pltpu.VMEM((2, GW, CH), jnp.int32),
            pltpu.VMEM((CH, D), jnp.float32),
            pltpu.VMEM((CH, D), jnp.float32),
            pltpu.VMEM_SHARED((NPAD, D), jnp.float32),
            pltpu.SemaphoreType.DMA,
            pltpu.SemaphoreType.DMA,
            pltpu.SemaphoreType.DMA,
        ],
    )
    def aggr_kernel(hp_hbm, src_hbm, dst_hbm, zer_hbm, out_hbm,
                    src_v, dstw, b0, b1, acc_sh, s0, s1, sd):
        c = lax.axis_index("c")
        s = lax.axis_index("s")
        w = c * NS + s
        sl = pl.ds(s * RPT, RPT)
        bufs = (b0, b1)
        sems = (s0, s1)
        ngrp = rpw // GW
        pltpu.sync_copy(zer_hbm, acc_sh.at[sl])
        pltpu.sync_copy(src_hbm.at[pl.ds(w * rpw, rpw)], src_v)
        pltpu.sync_copy(dst_hbm.at[pl.ds(w * rpw, GW)], dstw.at[0])
        plsc.subcore_barrier()

        for b in range(NBUF):
            pltpu.async_copy(hp_hbm.at[src_v.at[b]], bufs[b], sems[b])

        def outer(g, carry):
            cur = lax.rem(g, 2)
            nxt = jnp.minimum(g + 1, ngrp - 1)
            pltpu.async_copy(
                dst_hbm.at[pl.ds(w * rpw + nxt * GW, GW)],
                dstw.at[lax.rem(g + 1, 2)], sd)
            dcur = dstw.at[cur]
            for jj in range(GW):
                b = jj % NBUF
                j = g * GW + jj
                pltpu.make_async_copy(
                    hp_hbm.at[src_v.at[0]], bufs[b], sems[b]).wait()
                pltpu.sync_copy(bufs[b], acc_sh.at[dcur.at[jj]], add=True)
                jn = jnp.minimum(j + NBUF, rpw - 1)
                pltpu.async_copy(hp_hbm.at[src_v.at[jn]], bufs[b], sems[b])
            pltpu.make_async_copy(
                dst_hbm.at[pl.ds(w * rpw, GW)], dstw.at[0], sd).wait()
            return carry

        lax.fori_loop(0, ngrp, outer, 0)
        for b in range(NBUF):
            pltpu.make_async_copy(
                hp_hbm.at[src_v.at[0]], bufs[b], sems[b]).wait()
        plsc.subcore_barrier()
        pltpu.sync_copy(acc_sh.at[sl], out_hbm.at[c].at[sl])

    return aggr_kernel


def _tc_stage1(x_ref, w_ref, deg_ref, hp_ref, dinv_ref):
    deg = deg_ref[0, :N, 0:1] + deg_ref[1, :N, 0:1]
    dinv = lax.rsqrt(deg + 1.0)
    h = jnp.dot(x_ref[...], w_ref[...], preferred_element_type=jnp.float32)
    hp_ref[...] = h * dinv
    dinv_ref[...] = dinv


def _tc_mid(acc_ref, hp_ref, dinv_ref, b_ref, g_ref, be_ref, w_ref, out_ref):
    accsum = acc_ref[0, :N, :] + acc_ref[1, :N, :] + hp_ref[...]
    pre = accsum * dinv_ref[...] + b_ref[...]
    mu = jnp.mean(pre, axis=0, keepdims=True)
    var = jnp.mean((pre - mu) ** 2, axis=0, keepdims=True)
    y = jnp.maximum(g_ref[...] * (pre - mu) * lax.rsqrt(var + 1e-5)
                    + be_ref[...], 0.0)
    out_ref[...] = jnp.dot(y, w_ref[...],
                           preferred_element_type=jnp.float32) * dinv_ref[...]


def _tc_fin(acc_ref, hp_ref, dinv_ref, b_ref, batch_ref, wl_ref, bl_ref,
            out_ref):
    pre = (acc_ref[0, :N, :] + acc_ref[1, :N, :] + hp_ref[...]) \
        * dinv_ref[...] + b_ref[...]
    h = jnp.maximum(pre, 0.0)
    seg = lax.broadcasted_iota(jnp.int32, (G, N), 0)
    onehot = (seg == batch_ref[...]).astype(jnp.float32)
    pooled = jnp.dot(onehot, h, preferred_element_type=jnp.float32)
    counts = jnp.sum(onehot, axis=1, keepdims=True)
    pooled = pooled / jnp.maximum(counts, 1.0)
    out_ref[...] = jnp.dot(pooled, wl_ref[...],
                           preferred_element_type=jnp.float32) + bl_ref[...]


_f32 = jnp.float32

_stage1 = pl.pallas_call(
    _tc_stage1,
    out_shape=(jax.ShapeDtypeStruct((N, D), _f32),
               jax.ShapeDtypeStruct((N, 1), _f32)),
)

def _mid_call(acc, hp, dinv, b, g, be, w):
    return pl.pallas_call(
        _tc_mid,
        out_shape=jax.ShapeDtypeStruct((N, D), _f32),
    )(acc, hp, dinv, b, g, be, w)


def _fin_call(acc, hp, dinv, b, batch2d, wl, bl):
    return pl.pallas_call(
        _tc_fin,
        out_shape=jax.ShapeDtypeStruct((G, 1), _f32),
    )(acc, hp, dinv, b, batch2d, wl, bl)


def kernel(x, edge_index, batch, W1, b1, g1, be1, W2, b2, g2, be2, W3, b3,
           Wl, bl):
    E = edge_index.shape[1]
    rpw = -(-E // (NW * CH))
    rpw = -(-rpw // 8) * 8
    epad = NW * rpw * CH
    pad = epad - E
    src = jnp.concatenate([edge_index[0], jnp.zeros((pad,), jnp.int32)])
    trash = N + jnp.arange(pad, dtype=jnp.int32) % (NPAD - N)
    dst = jnp.concatenate([edge_index[1], trash])
    src2d = src.reshape(NW * rpw, CH)
    dst2d = dst.reshape(NW * rpw, CH)
    zer = jnp.zeros((RPT, D), _f32)

    deg = _make_deg(rpw)(dst2d, jnp.ones((CH, D), _f32), zer)

    aggr = _make_aggr(rpw)

    hp1, dinv = _stage1(x, W1, deg)
    acc1 = aggr(hp1, src2d, dst2d, zer)
    hp2 = _mid_call(acc1, hp1, dinv, b1.reshape(1, D), g1.reshape(1, D),
                    be1.reshape(1, D), W2)
    acc2 = aggr(hp2, src2d, dst2d, zer)
    hp3 = _mid_call(acc2, hp2, dinv, b2.reshape(1, D), g2.reshape(1, D),
                    be2.reshape(1, D), W3)
    acc3 = aggr(hp3, src2d, dst2d, zer)
    return _fin_call(acc3, hp3, dinv, b3.reshape(1, D), batch.reshape(1, N),
                     Wl, bl.reshape(1, 1))


# speedup vs baseline: 25.2116x; 3.2230x over previous
import functools

import jax
import jax.numpy as jnp
from jax import lax
from jax.experimental import pallas as pl
from jax.experimental.pallas import tpu as pltpu
from jax.experimental.pallas import tpu_sc as plsc

N = 10000
D = 128
G = 64
NC = 2
NS = 16
L = 16
NW = NC * NS
CH = 128
NPAD = 10240
RPT = NPAD // NS

_mesh = plsc.VectorSubcoreMesh(core_axis_name="c", subcore_axis_name="s")


def _make_deg(rpw):
    @functools.partial(
        pl.kernel,
        out_type=jax.ShapeDtypeStruct((NC, NPAD, D), jnp.float32),
        mesh=_mesh,
        scratch_types=[
            pltpu.VMEM((rpw, CH), jnp.int32),
            pltpu.VMEM((CH, D), jnp.float32),
            pltpu.VMEM_SHARED((NPAD, D), jnp.float32),
        ],
    )
    def deg_kernel(dst_hbm, ones_hbm, zer_hbm, out_hbm, dst_v, ones_v, acc_sh):
        c = lax.axis_index("c")
        s = lax.axis_index("s")
        w = c * NS + s
        sl = pl.ds(s * RPT, RPT)
        pltpu.sync_copy(zer_hbm, acc_sh.at[sl])
        pltpu.sync_copy(ones_hbm, ones_v)
        pltpu.sync_copy(dst_hbm.at[pl.ds(w * rpw, rpw)], dst_v)
        plsc.subcore_barrier()

        def body(j, carry):
            pltpu.sync_copy(ones_v, acc_sh.at[dst_v.at[j]], add=True)
            return carry

        lax.fori_loop(0, rpw, body, 0)
        plsc.subcore_barrier()
        pltpu.sync_copy(acc_sh.at[sl], out_hbm.at[c].at[sl])

    return deg_kernel


NBUF = 2
GW = 8


def _make_aggr(rpw):
    @functools.partial(
        pl.kernel,
        out_type=jax.ShapeDtypeStruct((NC, NPAD, D), jnp.float32),
        mesh=_mesh,
        scratch_types=[
            pltpu.VMEM((rpw, CH), jnp.int32),
            pltpu.VMEM((2, GW, CH), jnp.int32),
            pltpu.VMEM((CH, D), jnp.float32),
            pltpu.VMEM((CH, D), jnp.float32),
            pltpu.VMEM_SHARED((NPAD, D), jnp.float32),
            pltpu.SemaphoreType.DMA,
            pltpu.SemaphoreType.DMA,
            pltpu.SemaphoreType.DMA,
        ],
    )
    def aggr_kernel(hp_hbm, src_hbm, dst_hbm, zer_hbm, out_hbm,
                    src_v, dstw, b0, b1, acc_sh, s0, s1, sd):
        c = lax.axis_index("c")
        s = lax.axis_index("s")
        w = c * NS + s
        sl = pl.ds(s * RPT, RPT)
        bufs = (b0, b1)
        sems = (s0, s1)
        ngrp = rpw // GW
        pltpu.sync_copy(zer_hbm, acc_sh.at[sl])
        pltpu.sync_copy(src_hbm.at[pl.ds(w * rpw, rpw)], src_v)
        pltpu.sync_copy(dst_hbm.at[pl.ds(w * rpw, GW)], dstw.at[0])
        plsc.subcore_barrier()

        for b in range(NBUF):
            pltpu.async_copy(hp_hbm.at[src_v.at[b]], bufs[b], sems[b])

        def outer(g, carry):
            cur = lax.rem(g, 2)
            nxt = jnp.minimum(g + 1, ngrp - 1)
            pltpu.async_copy(
                dst_hbm.at[pl.ds(w * rpw + nxt * GW, GW)],
                dstw.at[lax.rem(g + 1, 2)], sd)
            dcur = dstw.at[cur]
            for jj in range(GW):
                b = jj % NBUF
                j = g * GW + jj
                pltpu.make_async_copy(
                    hp_hbm.at[src_v.at[0]], bufs[b], sems[b]).wait()
                pltpu.sync_copy(bufs[b], acc_sh.at[dcur.at[jj]], add=True)
                jn = jnp.minimum(j + NBUF, rpw - 1)
                pltpu.async_copy(hp_hbm.at[src_v.at[jn]], bufs[b], sems[b])
            pltpu.make_async_copy(
                dst_hbm.at[pl.ds(w * rpw, GW)], dstw.at[0], sd).wait()
            return carry

        lax.fori_loop(0, ngrp, outer, 0)
        for b in range(NBUF):
            pltpu.make_async_copy(
                hp_hbm.at[src_v.at[0]], bufs[b], sems[b]).wait()
        plsc.subcore_barrier()
        pltpu.sync_copy(acc_sh.at[sl], out_hbm.at[c].at[sl])

    return aggr_kernel


def _tc_stage1(x_ref, w_ref, deg_ref, hp_ref, dinv_ref):
    deg = deg_ref[0, :N, 0:1] + deg_ref[1, :N, 0:1]
    dinv = lax.rsqrt(deg + 1.0)
    h = jnp.dot(x_ref[...], w_ref[...], preferred_element_type=jnp.float32)
    hp_ref[...] = h * dinv
    dinv_ref[...] = dinv


def _tc_mid(acc_ref, hp_ref, dinv_ref, b_ref, g_ref, be_ref, w_ref, out_ref):
    accsum = acc_ref[0, :N, :] + acc_ref[1, :N, :] + hp_ref[...]
    pre = accsum * dinv_ref[...] + b_ref[...]
    mu = jnp.mean(pre, axis=0, keepdims=True)
    var = jnp.mean((pre - mu) ** 2, axis=0, keepdims=True)
    y = jnp.maximum(g_ref[...] * (pre - mu) * lax.rsqrt(var + 1e-5)
                    + be_ref[...], 0.0)
    out_ref[...] = jnp.dot(y, w_ref[...],
                           preferred_element_type=jnp.float32) * dinv_ref[...]


def _tc_fin(acc_ref, hp_ref, dinv_ref, b_ref, batch_ref, wl_ref, bl_ref,
            out_ref):
    pre = (acc_ref[0, :N, :] + acc_ref[1, :N, :] + hp_ref[...]) \
        * dinv_ref[...] + b_ref[...]
    h = jnp.maximum(pre, 0.0)
    seg = lax.broadcasted_iota(jnp.int32, (G, N), 0)
    onehot = (seg == batch_ref[...]).astype(jnp.float32)
    pooled = jnp.dot(onehot, h, preferred_element_type=jnp.float32)
    counts = jnp.sum(onehot, axis=1, keepdims=True)
    pooled = pooled / jnp.maximum(counts, 1.0)
    out_ref[...] = jnp.dot(pooled, wl_ref[...],
                           preferred_element_type=jnp.float32) + bl_ref[...]


_f32 = jnp.float32

_stage1 = pl.pallas_call(
    _tc_stage1,
    out_shape=(jax.ShapeDtypeStruct((N, D), _f32),
               jax.ShapeDtypeStruct((N, 1), _f32)),
)

def _mid_call(acc, hp, dinv, b, g, be, w):
    return pl.pallas_call(
        _tc_mid,
        out_shape=jax.ShapeDtypeStruct((N, D), _f32),
    )(acc, hp, dinv, b, g, be, w)


def _fin_call(acc, hp, dinv, b, batch2d, wl, bl):
    return pl.pallas_call(
        _tc_fin,
        out_shape=jax.ShapeDtypeStruct((G, 1), _f32),
    )(acc, hp, dinv, b, batch2d, wl, bl)


def kernel(x, edge_index, batch, W1, b1, g1, be1, W2, b2, g2, be2, W3, b3,
           Wl, bl):
    E = edge_index.shape[1]
    rpw = -(-E // (NW * CH))
    rpw = -(-rpw // 8) * 8
    epad = NW * rpw * CH
    pad = epad - E
    spread = jnp.arange(pad, dtype=jnp.int32) * 37 % N
    src = jnp.concatenate([edge_index[0], spread])
    trash = N + jnp.arange(pad, dtype=jnp.int32) % (NPAD - N)
    dst = jnp.concatenate([edge_index[1], trash])
    src2d = src.reshape(NW * rpw, CH)
    dst2d = dst.reshape(NW * rpw, CH)
    zer = jnp.zeros((RPT, D), _f32)

    deg = _make_deg(rpw)(dst2d, jnp.ones((CH, D), _f32), zer)

    aggr = _make_aggr(rpw)

    hp1, dinv = _stage1(x, W1, deg)
    acc1 = aggr(hp1, src2d, dst2d, zer)
    hp2 = _mid_call(acc1, hp1, dinv, b1.reshape(1, D), g1.reshape(1, D),
                    be1.reshape(1, D), W2)
    acc2 = aggr(hp2, src2d, dst2d, zer)
    hp3 = _mid_call(acc2, hp2, dinv, b2.reshape(1, D), g2.reshape(1, D),
                    be2.reshape(1, D), W3)
    acc3 = aggr(hp3, src2d, dst2d, zer)
    return _fin_call(acc3, hp3, dinv, b3.reshape(1, D), batch.reshape(1, N),
                     Wl, bl.reshape(1, 1))


# speedup vs baseline: 27.3665x; 1.0855x over previous
import functools

import jax
import jax.numpy as jnp
from jax import lax
from jax.experimental import pallas as pl
from jax.experimental.pallas import tpu as pltpu
from jax.experimental.pallas import tpu_sc as plsc

N = 10000
D = 128
G = 64
NC = 2
NS = 16
L = 16
NW = NC * NS
CH = 128
NPAD = 10240
RPT = NPAD // NS

_mesh = plsc.VectorSubcoreMesh(core_axis_name="c", subcore_axis_name="s")


DW = 32


def _make_deg(rpw):
    @functools.partial(
        pl.kernel,
        out_type=jax.ShapeDtypeStruct((NC, NPAD, DW), jnp.float32),
        mesh=_mesh,
        scratch_types=[
            pltpu.VMEM((rpw, CH), jnp.int32),
            pltpu.VMEM((CH, DW), jnp.float32),
            pltpu.VMEM_SHARED((NPAD, DW), jnp.float32),
        ],
    )
    def deg_kernel(dst_hbm, ones_hbm, zer_hbm, out_hbm, dst_v, ones_v, acc_sh):
        c = lax.axis_index("c")
        s = lax.axis_index("s")
        w = c * NS + s
        sl = pl.ds(s * RPT, RPT)
        pltpu.sync_copy(zer_hbm, acc_sh.at[sl])
        pltpu.sync_copy(ones_hbm, ones_v)
        pltpu.sync_copy(dst_hbm.at[pl.ds(w * rpw, rpw)], dst_v)
        plsc.subcore_barrier()

        def body(j, carry):
            pltpu.sync_copy(ones_v, acc_sh.at[dst_v.at[j]], add=True)
            return carry

        lax.fori_loop(0, rpw, body, 0)
        plsc.subcore_barrier()
        pltpu.sync_copy(acc_sh.at[sl], out_hbm.at[c].at[sl])

    return deg_kernel


NBUF = 2
GW = 8


def _make_aggr(rpw):
    @functools.partial(
        pl.kernel,
        out_type=jax.ShapeDtypeStruct((NC, NPAD, D), jnp.float32),
        mesh=_mesh,
        scratch_types=[
            pltpu.VMEM((rpw, CH), jnp.int32),
            pltpu.VMEM((2, GW, CH), jnp.int32),
            pltpu.VMEM((CH, D), jnp.float32),
            pltpu.VMEM((CH, D), jnp.float32),
            pltpu.VMEM_SHARED((NPAD, D), jnp.float32),
            pltpu.SemaphoreType.DMA,
            pltpu.SemaphoreType.DMA,
            pltpu.SemaphoreType.DMA,
        ],
    )
    def aggr_kernel(hp_hbm, src_hbm, dst_hbm, zer_hbm, out_hbm,
                    src_v, dstw, b0, b1, acc_sh, s0, s1, sd):
        c = lax.axis_index("c")
        s = lax.axis_index("s")
        w = c * NS + s
        sl = pl.ds(s * RPT, RPT)
        bufs = (b0, b1)
        sems = (s0, s1)
        ngrp = rpw // GW
        pltpu.sync_copy(zer_hbm, acc_sh.at[sl])
        pltpu.sync_copy(src_hbm.at[pl.ds(w * rpw, rpw)], src_v)
        pltpu.sync_copy(dst_hbm.at[pl.ds(w * rpw, GW)], dstw.at[0])
        plsc.subcore_barrier()

        for b in range(NBUF):
            pltpu.async_copy(hp_hbm.at[src_v.at[b]], bufs[b], sems[b])

        def outer(g, carry):
            cur = lax.rem(g, 2)
            nxt = jnp.minimum(g + 1, ngrp - 1)
            pltpu.async_copy(
                dst_hbm.at[pl.ds(w * rpw + nxt * GW, GW)],
                dstw.at[lax.rem(g + 1, 2)], sd)
            dcur = dstw.at[cur]
            for jj in range(GW):
                b = jj % NBUF
                j = g * GW + jj
                pltpu.make_async_copy(
                    hp_hbm.at[src_v.at[0]], bufs[b], sems[b]).wait()
                pltpu.sync_copy(bufs[b], acc_sh.at[dcur.at[jj]], add=True)
                jn = jnp.minimum(j + NBUF, rpw - 1)
                pltpu.async_copy(hp_hbm.at[src_v.at[jn]], bufs[b], sems[b])
            pltpu.make_async_copy(
                dst_hbm.at[pl.ds(w * rpw, GW)], dstw.at[0], sd).wait()
            return carry

        lax.fori_loop(0, ngrp, outer, 0)
        for b in range(NBUF):
            pltpu.make_async_copy(
                hp_hbm.at[src_v.at[0]], bufs[b], sems[b]).wait()
        plsc.subcore_barrier()
        pltpu.sync_copy(acc_sh.at[sl], out_hbm.at[c].at[sl])

    return aggr_kernel


def _tc_stage1(x_ref, w_ref, deg_ref, hp_ref, dinv_ref):
    deg = deg_ref[0, :N, 0:1] + deg_ref[1, :N, 0:1]
    dinv = lax.rsqrt(deg + 1.0)
    h = jnp.dot(x_ref[...], w_ref[...], preferred_element_type=jnp.float32)
    hp_ref[...] = h * dinv
    dinv_ref[...] = dinv


def _tc_mid(acc_ref, hp_ref, dinv_ref, b_ref, g_ref, be_ref, w_ref, out_ref):
    accsum = acc_ref[0, :N, :] + acc_ref[1, :N, :] + hp_ref[...]
    pre = accsum * dinv_ref[...] + b_ref[...]
    mu = jnp.mean(pre, axis=0, keepdims=True)
    var = jnp.mean((pre - mu) ** 2, axis=0, keepdims=True)
    y = jnp.maximum(g_ref[...] * (pre - mu) * lax.rsqrt(var + 1e-5)
                    + be_ref[...], 0.0)
    out_ref[...] = jnp.dot(y, w_ref[...],
                           preferred_element_type=jnp.float32) * dinv_ref[...]


def _tc_fin(acc_ref, hp_ref, dinv_ref, b_ref, batch_ref, wl_ref, bl_ref,
            out_ref):
    pre = (acc_ref[0, :N, :] + acc_ref[1, :N, :] + hp_ref[...]) \
        * dinv_ref[...] + b_ref[...]
    h = jnp.maximum(pre, 0.0)
    seg = lax.broadcasted_iota(jnp.int32, (G, N), 0)
    onehot = (seg == batch_ref[...]).astype(jnp.float32)
    pooled = jnp.dot(onehot, h, preferred_element_type=jnp.float32)
    counts = jnp.sum(onehot, axis=1, keepdims=True)
    pooled = pooled / jnp.maximum(counts, 1.0)
    out_ref[...] = jnp.dot(pooled, wl_ref[...],
                           preferred_element_type=jnp.float32) + bl_ref[...]


_f32 = jnp.float32

_stage1 = pl.pallas_call(
    _tc_stage1,
    out_shape=(jax.ShapeDtypeStruct((N, D), _f32),
               jax.ShapeDtypeStruct((N, 1), _f32)),
)

def _mid_call(acc, hp, dinv, b, g, be, w):
    return pl.pallas_call(
        _tc_mid,
        out_shape=jax.ShapeDtypeStruct((N, D), _f32),
    )(acc, hp, dinv, b, g, be, w)


def _fin_call(acc, hp, dinv, b, batch2d, wl, bl):
    return pl.pallas_call(
        _tc_fin,
        out_shape=jax.ShapeDtypeStruct((G, 1), _f32),
    )(acc, hp, dinv, b, batch2d, wl, bl)


def kernel(x, edge_index, batch, W1, b1, g1, be1, W2, b2, g2, be2, W3, b3,
           Wl, bl):
    E = edge_index.shape[1]
    rpw = -(-E // (NW * CH))
    rpw = -(-rpw // 8) * 8
    epad = NW * rpw * CH
    pad = epad - E
    spread = jnp.arange(pad, dtype=jnp.int32) * 37 % N
    src = jnp.concatenate([edge_index[0], spread])
    trash = N + jnp.arange(pad, dtype=jnp.int32) % (NPAD - N)
    dst = jnp.concatenate([edge_index[1], trash])
    src2d = src.reshape(NW * rpw, CH)
    dst2d = dst.reshape(NW * rpw, CH)
    zer = jnp.zeros((RPT, D), _f32)

    deg = _make_deg(rpw)(dst2d, jnp.ones((CH, DW), _f32),
                         jnp.zeros((RPT, DW), _f32))

    aggr = _make_aggr(rpw)

    hp1, dinv = _stage1(x, W1, deg)
    acc1 = aggr(hp1, src2d, dst2d, zer)
    hp2 = _mid_call(acc1, hp1, dinv, b1.reshape(1, D), g1.reshape(1, D),
                    be1.reshape(1, D), W2)
    acc2 = aggr(hp2, src2d, dst2d, zer)
    hp3 = _mid_call(acc2, hp2, dinv, b2.reshape(1, D), g2.reshape(1, D),
                    be2.reshape(1, D), W3)
    acc3 = aggr(hp3, src2d, dst2d, zer)
    return _fin_call(acc3, hp3, dinv, b3.reshape(1, D), batch.reshape(1, N),
                     Wl, bl.reshape(1, 1))
